# baseline (speedup 1.0000x reference)
import functools

import jax
import jax.numpy as jnp
from jax import lax
from jax.experimental import pallas as pl
from jax.experimental.pallas import tpu as pltpu
from jax.experimental.pallas import tpu_sc as plsc

_N = 10000
_E = 320000
_D = 128
_NC = 2
_NS = 16
_NW = _NC * _NS
_EPT = _E // _NW
_CH = 128
_NCHP = 80
_EPP = _CH * _NCHP
_NBUF = 3
_RPT = _N // _NS
_BLK = 1000
_NB = _N // _BLK

_SC_PARAMS = pltpu.CompilerParams(needs_layout_passes=False,
                                  use_tc_tiling_on_sc=False)


@functools.lru_cache(maxsize=None)
def _sc_mesh():
    return plsc.VectorSubcoreMesh(core_axis_name="c", subcore_axis_name="s",
                                  num_cores=_NC, num_subcores=_NS)


def _sc_degree(dstT, wT):
    @functools.partial(
        pl.kernel,
        out_type=jax.ShapeDtypeStruct((_NW, _RPT, 16), jnp.float32),
        mesh=_sc_mesh(),
        compiler_params=_SC_PARAMS,
        scratch_types=[
            pltpu.VMEM_SHARED((_N, 16), jnp.float32),
            pltpu.VMEM((_NCHP, _CH), jnp.int32),
            pltpu.VMEM((_EPP,), jnp.float32),
            pltpu.VMEM((_CH, 16), jnp.float32),
        ],
    )
    def k(dst_hbm, w_hbm, out_hbm, acc, dstb, wb, stage):
        c = lax.axis_index("c")
        s = lax.axis_index("s")
        wid = c * _NS + s

        def _z(j, carry):
            stage[j] = jnp.zeros((16,), jnp.float32)
            return carry

        lax.fori_loop(0, _CH, _z, 0)
        row0 = s * _RPT
        for kk in range(_RPT // _CH):
            pltpu.sync_copy(stage, acc.at[pl.ds(row0 + kk * _CH, _CH)])
        rem = _RPT % _CH
        if rem:
            pltpu.sync_copy(stage.at[pl.ds(0, rem)],
                            acc.at[pl.ds(row0 + _RPT - rem, rem)])
        pltpu.sync_copy(dst_hbm.at[wid], dstb)
        pltpu.sync_copy(w_hbm.at[wid], wb)
        plsc.subcore_barrier()

        def _chunk(ci, carry):
            def _fill(j, c2):
                wj = plsc.load_gather(
                    wb, [jnp.full((16,), ci * _CH + j, jnp.int32)])
                stage[j] = wj
                return c2

            lax.fori_loop(0, _CH, _fill, 0, unroll=8)
            pltpu.sync_copy(stage, acc.at[dstb.at[ci]], add=True)
            return carry

        lax.fori_loop(0, _NCHP, _chunk, 0)
        plsc.subcore_barrier()
        pltpu.sync_copy(acc.at[pl.ds(row0, _RPT)], out_hbm.at[wid])

    return k(dstT, wT).reshape(_NC, _N, 16)


def _sc_scatter(g, srcT, dstT, wT):
    @functools.partial(
        pl.kernel,
        out_type=jax.ShapeDtypeStruct((_NW, _RPT, _D), jnp.float32),
        mesh=_sc_mesh(),
        compiler_params=_SC_PARAMS,
        scratch_types=[
            pltpu.VMEM_SHARED((_N, _D), jnp.float32),
            pltpu.VMEM((2, _CH), jnp.int32),
            pltpu.VMEM((2, _CH), jnp.int32),
            pltpu.VMEM((_EPP,), jnp.float32),
            pltpu.VMEM((2, _CH, _D), jnp.float32),
            [pltpu.SemaphoreType.DMA] * 2,
            [pltpu.SemaphoreType.DMA] * 2,
            [pltpu.SemaphoreType.DMA] * 2,
            [pltpu.SemaphoreType.DMA] * 2,
        ],
    )
    def k(g_hbm, src_hbm, dst_hbm, w_hbm, out_hbm,
          acc, sring, dring, wb, rows, gsems, ssems, sfs, dfs):
        c = lax.axis_index("c")
        s = lax.axis_index("s")
        wid = c * _NS + s

        def _z(j, carry):
            for t in range(_D // 16):
                rows[0, j, pl.ds(t * 16, 16)] = jnp.zeros((16,), jnp.float32)
            return carry

        lax.fori_loop(0, _CH, _z, 0)
        row0 = s * _RPT
        for kk in range(_RPT // _CH):
            pltpu.sync_copy(rows.at[0], acc.at[pl.ds(row0 + kk * _CH, _CH)])
        rem = _RPT % _CH
        if rem:
            pltpu.sync_copy(rows.at[0, pl.ds(0, rem)],
                            acc.at[pl.ds(row0 + _RPT - rem, rem)])
        pltpu.sync_copy(w_hbm.at[wid], wb)
        plsc.subcore_barrier()

        def _gather_wait(b):
            pltpu.make_async_copy(g_hbm.at[sring.at[0]], rows.at[b],
                                  gsems[b]).wait()

        def _scat_wait(b):
            pltpu.make_async_copy(rows.at[b], acc.at[dring.at[0]],
                                  ssems[b]).wait()

        pltpu.sync_copy(src_hbm.at[wid, 0], sring.at[0])
        pltpu.sync_copy(dst_hbm.at[wid, 0], dring.at[0])
        pltpu.async_copy(g_hbm.at[sring.at[0]], rows.at[0], gsems[0])

        def _slot(ci, b):
            b1 = (b + 1) % 2

            @pl.when(ci >= 1)
            def _():
                _scat_wait(b1)

            @pl.when(ci + 1 < _NCHP)
            def _():
                pltpu.async_copy(src_hbm.at[wid, ci + 1], sring.at[b1],
                                 sfs[b1])
                pltpu.async_copy(dst_hbm.at[wid, ci + 1], dring.at[b1],
                                 dfs[b1])

            _gather_wait(b)

            @pl.when(ci + 1 < _NCHP)
            def _():
                pltpu.make_async_copy(src_hbm.at[wid, 0], sring.at[0],
                                      sfs[b1]).wait()
                pltpu.async_copy(g_hbm.at[sring.at[b1]], rows.at[b1],
                                 gsems[b1])

            def _mul(j, c2):
                wj = plsc.load_gather(
                    wb, [jnp.full((16,), ci * _CH + j, jnp.int32)])
                for t in range(_D // 16):
                    rows[b, j, pl.ds(t * 16, 16)] = (
                        rows[b, j, pl.ds(t * 16, 16)] * wj)
                return c2

            lax.fori_loop(0, _CH, _mul, 0, unroll=8)

            @pl.when(ci >= 1)
            def _():
                pltpu.make_async_copy(dst_hbm.at[wid, 0], dring.at[0],
                                      dfs[b]).wait()

            pltpu.async_copy(rows.at[b], acc.at[dring.at[b]], ssems[b],
                             add=True)

        def _group(gr, carry):
            for b in range(2):
                _slot(gr * 2 + b, b)
            return carry

        lax.fori_loop(0, _NCHP // 2, _group, 0)
        _scat_wait((_NCHP - 1) % 2)
        plsc.subcore_barrier()
        pltpu.sync_copy(acc.at[pl.ds(row0, _RPT)], out_hbm.at[wid])

    return k(g, srcT, dstT, wT).reshape(_NC, _N, _D)


def _tc_matmul(x, W):
    def body(x_ref, w_ref, o_ref):
        o_ref[...] = jnp.dot(x_ref[...], w_ref[...],
                             preferred_element_type=jnp.float32)

    return pl.pallas_call(
        body,
        grid=(_NB,),
        in_specs=[pl.BlockSpec((_BLK, _D), lambda i: (i, 0)),
                  pl.BlockSpec((_D, _D), lambda i: (0, 0))],
        out_specs=pl.BlockSpec((_BLK, _D), lambda i: (i, 0)),
        out_shape=jax.ShapeDtypeStruct((_N, _D), jnp.float32),
    )(x, W)


def _tc_dis_g1(deg2, xw):
    def body(d_ref, xw_ref, dis_ref, g_ref):
        d = d_ref[0] + d_ref[1] + 1.0
        r = lax.rsqrt(d)
        sel = (lax.broadcasted_iota(jnp.int32, (16, _D), 0)
               == lax.broadcasted_iota(jnp.int32, (16, _D), 1) % 16
               ).astype(jnp.float32)
        dis_b = jnp.dot(r, sel, preferred_element_type=jnp.float32,
                        precision=lax.Precision.HIGHEST)
        dis_ref[...] = dis_b
        g_ref[...] = dis_b * xw_ref[...]

    return pl.pallas_call(
        body,
        grid=(_NB,),
        in_specs=[pl.BlockSpec((_NC, _BLK, 16), lambda i: (0, i, 0)),
                  pl.BlockSpec((_BLK, _D), lambda i: (i, 0))],
        out_specs=[pl.BlockSpec((_BLK, _D), lambda i: (i, 0)),
                   pl.BlockSpec((_BLK, _D), lambda i: (i, 0))],
        out_shape=[jax.ShapeDtypeStruct((_N, _D), jnp.float32),
                   jax.ShapeDtypeStruct((_N, _D), jnp.float32)],
    )(deg2, xw)


def _tc_bn_fused(S2, g, dis_b, b, gamma, beta, W2):
    def body(s_ref, g_ref, dis_ref, b_ref, ga_ref, be_ref, w2_ref,
             o_ref, st):
        i = pl.program_id(0)
        h = dis_ref[...] * (s_ref[0] + s_ref[1] + g_ref[...]) + b_ref[...]

        @pl.when(i == 0)
        def _():
            st[...] = jnp.zeros_like(st)

        @pl.when(i < _NB)
        def _():
            st[0:1] += jnp.sum(h, 0, keepdims=True)
            st[1:2] += jnp.sum(h * h, 0, keepdims=True)

        @pl.when(i >= _NB)
        def _():
            mu = st[0:1] / _N
            var = st[1:2] / _N - mu * mu
            y = (ga_ref[...] * (h - mu) * lax.rsqrt(var + 1e-5)
                 + be_ref[...])
            y = jnp.maximum(y, 0.0)
            o_ref[...] = dis_ref[...] * jnp.dot(
                y, w2_ref[...], preferred_element_type=jnp.float32)

    return pl.pallas_call(
        body,
        grid=(2 * _NB,),
        in_specs=[pl.BlockSpec((_NC, _BLK, _D), lambda i: (0, i % _NB, 0)),
                  pl.BlockSpec((_BLK, _D), lambda i: (i % _NB, 0)),
                  pl.BlockSpec((_BLK, _D), lambda i: (i % _NB, 0)),
                  pl.BlockSpec((1, _D), lambda i: (0, 0)),
                  pl.BlockSpec((1, _D), lambda i: (0, 0)),
                  pl.BlockSpec((1, _D), lambda i: (0, 0)),
                  pl.BlockSpec((_D, _D), lambda i: (0, 0))],
        out_specs=pl.BlockSpec((_BLK, _D), lambda i: (i % _NB, 0)),
        out_shape=jax.ShapeDtypeStruct((_N, _D), jnp.float32),
        scratch_shapes=[pltpu.VMEM((8, _D), jnp.float32)],
    )(S2, g, dis_b, b, gamma, beta, W2)


def _tc_bn_final(S2, g, dis_b, b, gamma, beta):
    def body(s_ref, g_ref, dis_ref, b_ref, ga_ref, be_ref, o_ref, st):
        i = pl.program_id(0)
        h = dis_ref[...] * (s_ref[0] + s_ref[1] + g_ref[...]) + b_ref[...]

        @pl.when(i == 0)
        def _():
            st[...] = jnp.zeros_like(st)

        @pl.when(i < _NB)
        def _():
            st[0:1] += jnp.sum(h, 0, keepdims=True)
            st[1:2] += jnp.sum(h * h, 0, keepdims=True)

        @pl.when(i >= _NB)
        def _():
            mu = st[0:1] / _N
            var = st[1:2] / _N - mu * mu
            o_ref[...] = (ga_ref[...] * (h - mu) * lax.rsqrt(var + 1e-5)
                          + be_ref[...])

    return pl.pallas_call(
        body,
        grid=(2 * _NB,),
        in_specs=[pl.BlockSpec((_NC, _BLK, _D), lambda i: (0, i % _NB, 0)),
                  pl.BlockSpec((_BLK, _D), lambda i: (i % _NB, 0)),
                  pl.BlockSpec((_BLK, _D), lambda i: (i % _NB, 0)),
                  pl.BlockSpec((1, _D), lambda i: (0, 0)),
                  pl.BlockSpec((1, _D), lambda i: (0, 0)),
                  pl.BlockSpec((1, _D), lambda i: (0, 0))],
        out_specs=pl.BlockSpec((_BLK, _D), lambda i: (i % _NB, 0)),
        out_shape=jax.ShapeDtypeStruct((_N, _D), jnp.float32),
        scratch_shapes=[pltpu.VMEM((8, _D), jnp.float32)],
    )(S2, g, dis_b, b, gamma, beta)


def _pad_tiles(a, fill, flat=False):
    a = a.reshape(_NW, _EPT)
    a = jnp.pad(a, ((0, 0), (0, _EPP - _EPT)), constant_values=fill)
    return a if flat else a.reshape(_NW, _NCHP, _CH)


def kernel(x, edge_index, edge_weight, W1, b1, gamma1, beta1,
           W2, b2, gamma2, beta2):
    src = _pad_tiles(edge_index[0], 0)
    dst = _pad_tiles(edge_index[1], 0)
    w = _pad_tiles(edge_weight, 0.0, flat=True)

    deg2 = _sc_degree(dst, w)
    xw = _tc_matmul(x, W1)
    dis_b, g1 = _tc_dis_g1(deg2, xw)

    S1 = _sc_scatter(g1, src, dst, w)
    g2 = _tc_bn_fused(S1, g1, dis_b, b1.reshape(1, _D),
                      gamma1.reshape(1, _D), beta1.reshape(1, _D), W2)

    S2 = _sc_scatter(g2, src, dst, w)
    return _tc_bn_final(S2, g2, dis_b, b2.reshape(1, _D),
                        gamma2.reshape(1, _D), beta2.reshape(1, _D))
